# baseline (speedup 1.0000x reference)
import functools

import jax
import jax.numpy as jnp
from jax import lax
from jax.experimental import pallas as pl
from jax.experimental.pallas import tpu as pltpu
from jax.experimental.pallas import tpu_sc as plsc

_NC = 2
_NS = 16
_NW = _NC * _NS
_L = 16


def _embedding_bag_sc(indices, table):
    B, A = indices.shape
    _, D = table.shape
    rpw = B // _NW
    nvec = D // _L
    assert rpw % 4 == 0 and rpw >= 8

    mesh = plsc.VectorSubcoreMesh(core_axis_name="c", subcore_axis_name="s")

    @functools.partial(
        pl.kernel,
        out_type=jax.ShapeDtypeStruct((B, D), jnp.float32),
        mesh=mesh,
        scratch_types=[
            pltpu.VMEM((A,), jnp.int32),
            pltpu.VMEM((A,), jnp.int32),
            pltpu.VMEM((A,), jnp.int32),
            pltpu.VMEM((A,), jnp.int32),
            pltpu.VMEM((A, D), jnp.float32),
            pltpu.VMEM((A, D), jnp.float32),
            pltpu.VMEM((rpw, D), jnp.float32),
            pltpu.SemaphoreType.DMA,
            pltpu.SemaphoreType.DMA,
            pltpu.SemaphoreType.DMA,
            pltpu.SemaphoreType.DMA,
            pltpu.SemaphoreType.DMA,
            pltpu.SemaphoreType.DMA,
        ],
    )
    def bag(idx_hbm, table_hbm, out_hbm, ib0, ib1, ib2, ib3, rows0, rows1,
            acc_v, si0, si1, si2, si3, sg0, sg1):
        wid = lax.axis_index("s") * _NC + lax.axis_index("c")
        base = wid * rpw

        def aidx(r, ib, si):
            pltpu.async_copy(idx_hbm.at[base + r], ib, si)

        def widx(ib, si):
            pltpu.make_async_copy(idx_hbm.at[base], ib, si).wait()

        def gat(ib, buf, sg):
            pltpu.async_copy(table_hbm.at[ib], buf, sg)

        def wgat(buf, sg):
            pltpu.make_async_copy(table_hbm.at[ib0], buf, sg).wait()

        def accumulate(r, buf):
            def body(i, accs):
                out = []
                for t in range(nvec):
                    sl = pl.ds(t * _L, _L)
                    out.append(accs[t] + (buf[2 * i, sl] + buf[2 * i + 1, sl]))
                return tuple(out)

            accs = lax.fori_loop(
                0, A // 2, body,
                tuple(jnp.zeros((_L,), jnp.float32) for _ in range(nvec)),
            )
            for j in range(A // 2 * 2, A):
                accs = tuple(
                    accs[t] + buf[j, pl.ds(t * _L, _L)] for t in range(nvec)
                )
            for t in range(nvec):
                acc_v[r, pl.ds(t * _L, _L)] = accs[t]

        def super_iter(r, tail):
            wgat(rows0, sg0)
            if not tail:
                aidx(r + 4, ib0, si0)
            accumulate(r, rows0)
            widx(ib2, si2)
            gat(ib2, rows0, sg0)
            wgat(rows1, sg1)
            if not tail:
                aidx(r + 5, ib1, si1)
            accumulate(r + 1, rows1)
            widx(ib3, si3)
            gat(ib3, rows1, sg1)
            wgat(rows0, sg0)
            if not tail:
                aidx(r + 6, ib2, si2)
            accumulate(r + 2, rows0)
            if not tail:
                widx(ib0, si0)
                gat(ib0, rows0, sg0)
            wgat(rows1, sg1)
            if not tail:
                aidx(r + 7, ib3, si3)
            accumulate(r + 3, rows1)
            if not tail:
                widx(ib1, si1)
                gat(ib1, rows1, sg1)

        aidx(0, ib0, si0)
        aidx(1, ib1, si1)
        aidx(2, ib2, si2)
        aidx(3, ib3, si3)
        widx(ib0, si0)
        gat(ib0, rows0, sg0)
        widx(ib1, si1)
        gat(ib1, rows1, sg1)

        @pl.loop(0, rpw - 4, step=4)
        def _(r):
            super_iter(r, False)

        super_iter(rpw - 4, True)
        pltpu.sync_copy(acc_v, out_hbm.at[pl.ds(base, rpw)])

    return bag(indices, table)


def _mlp_tc(sums, b_enc, W1, b1, W2, b2, W3, b3):
    B = sums.shape[0]

    def body(x_ref, be_ref, w1_ref, b1_ref, w2_ref, b2_ref, w3_ref, b3_ref,
             o_ref):
        pr = jax.lax.Precision.DEFAULT
        c1 = jnp.clip(x_ref[...] + be_ref[...], 0.0, 1.0)
        c2 = jnp.clip(jnp.dot(c1, w1_ref[...], precision=pr) + b1_ref[...],
                      0.0, 1.0)
        c3 = jnp.clip(jnp.dot(c2, w2_ref[...], precision=pr) + b2_ref[...],
                      0.0, 1.0)
        l4 = jnp.dot(c3, w3_ref[...], precision=pr) + b3_ref[...]
        lane = lax.broadcasted_iota(jnp.int32, l4.shape, 1)
        o_ref[...] = jnp.where(lane == 0,
                               1000.0 * jnp.minimum(l4, 0.0),
                               jnp.maximum(l4, 0.0))

    return pl.pallas_call(
        body,
        out_shape=jax.ShapeDtypeStruct((B, 2), jnp.float32),
    )(sums, b_enc.reshape(1, -1), W1, b1.reshape(1, -1),
      W2, b2.reshape(1, -1), W3, b3.reshape(1, -1))


def kernel(indices, W_enc, b_enc, W1, b1, W2, b2, W3, b3):
    half = indices.shape[0] // 2
    s0 = _embedding_bag_sc(indices[:half], W_enc)
    s1 = _embedding_bag_sc(indices[half:], W_enc)
    o0 = _mlp_tc(s0, b_enc, W1, b1, W2, b2, W3, b3)
    o1 = _mlp_tc(s1, b_enc, W1, b1, W2, b2, W3, b3)
    return jnp.concatenate([o0, o1], axis=0)


# speedup vs baseline: 1.0981x; 1.0981x over previous
import functools

import jax
import jax.numpy as jnp
from jax import lax
from jax.experimental import pallas as pl
from jax.experimental.pallas import tpu as pltpu
from jax.experimental.pallas import tpu_sc as plsc

_NC = 2
_NS = 16
_NW = _NC * _NS
_L = 16


def _embedding_bag_sc(indices, table):
    B, A = indices.shape
    _, D = table.shape
    rpw = B // _NW
    nvec = D // _L
    assert rpw % 4 == 0 and rpw >= 8

    mesh = plsc.VectorSubcoreMesh(core_axis_name="c", subcore_axis_name="s")

    @functools.partial(
        pl.kernel,
        out_type=jax.ShapeDtypeStruct((B, D), jnp.float32),
        mesh=mesh,
        scratch_types=[
            pltpu.VMEM((A,), jnp.int32),
            pltpu.VMEM((A,), jnp.int32),
            pltpu.VMEM((A,), jnp.int32),
            pltpu.VMEM((A,), jnp.int32),
            pltpu.VMEM((A, D), jnp.float32),
            pltpu.VMEM((A, D), jnp.float32),
            pltpu.VMEM((rpw, D), jnp.float32),
            pltpu.SemaphoreType.DMA,
            pltpu.SemaphoreType.DMA,
            pltpu.SemaphoreType.DMA,
            pltpu.SemaphoreType.DMA,
            pltpu.SemaphoreType.DMA,
            pltpu.SemaphoreType.DMA,
        ],
    )
    def bag(idx_hbm, table_hbm, out_hbm, ib0, ib1, ib2, ib3, rows0, rows1,
            acc_v, si0, si1, si2, si3, sg0, sg1):
        wid = lax.axis_index("s") * _NC + lax.axis_index("c")
        base = wid * rpw

        def aidx(r, ib, si):
            pltpu.async_copy(idx_hbm.at[base + r], ib, si)

        def widx(ib, si):
            pltpu.make_async_copy(idx_hbm.at[base], ib, si).wait()

        def gat(ib, buf, sg):
            pltpu.async_copy(table_hbm.at[ib], buf, sg)

        def wgat(buf, sg):
            pltpu.make_async_copy(table_hbm.at[ib0], buf, sg).wait()

        def accumulate(r, buf):
            def body(i, accs):
                out = []
                for t in range(nvec):
                    sl = pl.ds(t * _L, _L)
                    out.append(accs[t] + (buf[2 * i, sl] + buf[2 * i + 1, sl]))
                return tuple(out)

            accs = lax.fori_loop(
                0, A // 2, body,
                tuple(jnp.zeros((_L,), jnp.float32) for _ in range(nvec)),
            )
            for j in range(A // 2 * 2, A):
                accs = tuple(
                    accs[t] + buf[j, pl.ds(t * _L, _L)] for t in range(nvec)
                )
            for t in range(nvec):
                acc_v[r, pl.ds(t * _L, _L)] = accs[t]

        def super_iter(r, tail):
            wgat(rows0, sg0)
            if not tail:
                aidx(r + 4, ib0, si0)
            accumulate(r, rows0)
            widx(ib2, si2)
            gat(ib2, rows0, sg0)
            wgat(rows1, sg1)
            if not tail:
                aidx(r + 5, ib1, si1)
            accumulate(r + 1, rows1)
            widx(ib3, si3)
            gat(ib3, rows1, sg1)
            wgat(rows0, sg0)
            if not tail:
                aidx(r + 6, ib2, si2)
            accumulate(r + 2, rows0)
            if not tail:
                widx(ib0, si0)
                gat(ib0, rows0, sg0)
            wgat(rows1, sg1)
            if not tail:
                aidx(r + 7, ib3, si3)
            accumulate(r + 3, rows1)
            if not tail:
                widx(ib1, si1)
                gat(ib1, rows1, sg1)

        aidx(0, ib0, si0)
        aidx(1, ib1, si1)
        aidx(2, ib2, si2)
        aidx(3, ib3, si3)
        widx(ib0, si0)
        gat(ib0, rows0, sg0)
        widx(ib1, si1)
        gat(ib1, rows1, sg1)

        @pl.loop(0, rpw - 4, step=4)
        def _(r):
            super_iter(r, False)

        super_iter(rpw - 4, True)
        pltpu.sync_copy(acc_v, out_hbm.at[pl.ds(base, rpw)])

    return bag(indices, table)


def _mlp_tc(sums, b_enc, W1, b1, W2, b2, W3, b3):
    B = sums.shape[0]

    def body(x_ref, be_ref, w1_ref, b1_ref, w2_ref, b2_ref, w3_ref, b3_ref,
             o_ref):
        pr = jax.lax.Precision.DEFAULT
        c1 = jnp.clip(x_ref[...] + be_ref[...], 0.0, 1.0)
        c2 = jnp.clip(jnp.dot(c1, w1_ref[...], precision=pr) + b1_ref[...],
                      0.0, 1.0)
        c3 = jnp.clip(jnp.dot(c2, w2_ref[...], precision=pr) + b2_ref[...],
                      0.0, 1.0)
        l4 = jnp.dot(c3, w3_ref[...], precision=pr) + b3_ref[...]
        lane = lax.broadcasted_iota(jnp.int32, l4.shape, 1)
        o_ref[...] = jnp.where(lane == 0,
                               1000.0 * jnp.minimum(l4, 0.0),
                               jnp.maximum(l4, 0.0))

    return pl.pallas_call(
        body,
        out_shape=jax.ShapeDtypeStruct((B, 2), jnp.float32),
    )(sums, b_enc.reshape(1, -1), W1, b1.reshape(1, -1),
      W2, b2.reshape(1, -1), W3, b3.reshape(1, -1))


def kernel(indices, W_enc, b_enc, W1, b1, W2, b2, W3, b3):
    sums = _embedding_bag_sc(indices, W_enc)
    return _mlp_tc(sums, b_enc, W1, b1, W2, b2, W3, b3)
